# Initial kernel scaffold; baseline (speedup 1.0000x reference)
import jax
import jax.numpy as jnp
from jax.experimental import pallas as pl


def kernel(x, edge_index, edge_attr, batch, W1, b1, W2, b2, Wl, bl):
    raise NotImplementedError("write your pallas kernel here")


# speedup vs baseline: 11.4184x; 11.4184x over previous
import functools

import jax
import jax.numpy as jnp
from jax import lax
from jax.experimental import pallas as pl
from jax.experimental.pallas import tpu as pltpu
from jax.experimental.pallas import tpu_sc as plsc

_N = 10000
_E = 320000
_D = 128
_H = 64
_G = 64

_NC = 2
_NS = 16
_NW = _NC * _NS
_CHUNK = 128
_CHUNKS = -(-(_E // _NW) // _CHUNK)
_EPT = _CHUNKS * _CHUNK
_EPAD = _EPT * _NW

_ROWS_Z = 125
_NZ = _N // (_NS * _ROWS_Z)

_BLK = 2000
_HIGH = lax.Precision.HIGHEST


def _sc_deg_body(c_hbm, w_hbm, degp_hbm, cbuf, wbuf, degl):
    cid = lax.axis_index("c")
    sid = lax.axis_index("s")
    wid = cid * _NS + sid
    pltpu.sync_copy(c_hbm.at[wid], cbuf)
    pltpu.sync_copy(w_hbm.at[wid], wbuf)

    z16 = jnp.zeros((16,), jnp.float32)

    @pl.loop(0, _N // 16)
    def _(i):
        degl[i, :] = z16

    @pl.loop(0, _CHUNKS)
    def _(j):
        for k in range(_CHUNK // 16):
            cv = cbuf[j, pl.ds(k * 16, 16)]
            wv = wbuf[j, pl.ds(k * 16, 16)]
            plsc.addupdate_scatter(degl, [cv >> 4, cv & 15], wv)

    pltpu.sync_copy(degl, degp_hbm.at[wid])


@functools.lru_cache(maxsize=None)
def _sc_kernels():
    mesh = plsc.VectorSubcoreMesh(
        core_axis_name="c", subcore_axis_name="s",
        num_cores=_NC, num_subcores=_NS,
    )
    sc_deg = pl.kernel(
        _sc_deg_body,
        out_type=jax.ShapeDtypeStruct((_NW, _N // 16, 16), jnp.float32),
        mesh=mesh,
        scratch_types=[
            pltpu.VMEM((_CHUNKS, _CHUNK), jnp.int32),
            pltpu.VMEM((_CHUNKS, _CHUNK), jnp.float32),
            pltpu.VMEM((_N // 16, 16), jnp.float32),
        ],
        compiler_params=pltpu.CompilerParams(needs_layout_passes=False, use_tc_tiling_on_sc=False),
    )
    sc_conv = pl.kernel(
        _sc_conv_body,
        out_type=jax.ShapeDtypeStruct((_NC, _N, _H), jnp.float32),
        mesh=mesh,
        scratch_types=[
            pltpu.VMEM((_CHUNKS, _CHUNK), jnp.int32),
            pltpu.VMEM((_CHUNKS, _CHUNK), jnp.int32),
            pltpu.VMEM((_CHUNKS, _CHUNK), jnp.float32),
            pltpu.VMEM((_CHUNK, _H), jnp.float32),
            pltpu.VMEM((_ROWS_Z, _H), jnp.float32),
            pltpu.VMEM_SHARED((_N, _H), jnp.float32),
            pltpu.SemaphoreType.DMA,
        ],
        compiler_params=pltpu.CompilerParams(needs_layout_passes=False, use_tc_tiling_on_sc=False),
    )
    return sc_deg, sc_conv


def _sc_conv_body(u_hbm, r_hbm, c_hbm, w_hbm, accp_hbm,
                  rbuf, cbuf, wbuf, rows, tbuf, acc_sh, sem):
    cid = lax.axis_index("c")
    sid = lax.axis_index("s")
    wid = cid * _NS + sid
    pltpu.sync_copy(r_hbm.at[wid], rbuf)
    pltpu.sync_copy(c_hbm.at[wid], cbuf)
    pltpu.sync_copy(w_hbm.at[wid], wbuf)

    z16 = jnp.zeros((16,), jnp.float32)

    @pl.loop(0, _ROWS_Z)
    def _(i):
        for q in range(_H // 16):
            tbuf[i, pl.ds(q * 16, 16)] = z16

    base = sid * (_NZ * _ROWS_Z)
    for i in range(_NZ):
        pltpu.sync_copy(tbuf, acc_sh.at[pl.ds(base + i * _ROWS_Z, _ROWS_Z)])
    plsc.subcore_barrier()

    @pl.loop(0, _CHUNKS)
    def _(j):
        pltpu.async_copy(u_hbm.at[rbuf.at[j]], rows, sem).wait()

        @pl.loop(0, _CHUNK // 16)
        def _(k):
            wv = wbuf[j, pl.ds(k * 16, 16)]
            for e16 in range(16):
                s = wv[e16]
                e = k * 16 + e16
                for q in range(_H // 16):
                    rows[e, pl.ds(q * 16, 16)] = rows[e, pl.ds(q * 16, 16)] * s

        pltpu.sync_copy(rows, acc_sh.at[cbuf.at[j]], add=True)

    plsc.subcore_barrier()
    for i in range(_NZ):
        sl = pl.ds(base + i * _ROWS_Z, _ROWS_Z)
        pltpu.sync_copy(acc_sh.at[sl], tbuf)
        pltpu.sync_copy(tbuf, accp_hbm.at[cid, sl])


def _tc_pre_body(x_ref, w1_ref, degp_ref, xw_ref, u_ref, dis_ref):
    deg = 1.0 + jnp.sum(degp_ref[...], axis=1)
    dis = lax.rsqrt(deg)[:, None]
    xw = jnp.dot(x_ref[...], w1_ref[...],
                 preferred_element_type=jnp.float32, precision=_HIGH)
    xw_ref[...] = xw
    u_ref[...] = xw * dis
    dis_ref[...] = dis


def _tc_mid_body(accp_ref, xw1_ref, dis_ref, b1_ref, w2_ref, xw2_ref, u2_ref):
    dis = dis_ref[...]
    acc = accp_ref[0] + accp_ref[1]
    h1 = jnp.maximum(dis * (acc + dis * xw1_ref[...]) + b1_ref[...], 0.0)
    xw2 = jnp.dot(h1, w2_ref[...],
                  preferred_element_type=jnp.float32, precision=_HIGH)
    xw2_ref[...] = xw2
    u2_ref[...] = xw2 * dis


def _tc_post_body(accp_ref, xw2_ref, dis_ref, b2_ref, batch_ref, wl_ref,
                  bl_ref, out_ref, sums, cnt):
    step = pl.program_id(0)
    dis = dis_ref[...]
    acc = accp_ref[0] + accp_ref[1]
    h2 = dis * (acc + dis * xw2_ref[...]) + b2_ref[...]
    b = batch_ref[...][:, 0]
    gid = lax.broadcasted_iota(jnp.int32, (_G, _BLK), 0)
    m = (b[None, :] == gid).astype(jnp.float32)
    ps = jnp.dot(m, h2, preferred_element_type=jnp.float32, precision=_HIGH)
    pc = jnp.sum(m, axis=1, keepdims=True)

    @pl.when(step == 0)
    def _():
        sums[...] = ps
        cnt[...] = pc

    @pl.when(step > 0)
    def _():
        sums[...] += ps
        cnt[...] += pc

    @pl.when(step == pl.num_programs(0) - 1)
    def _():
        pooled = sums[...] / jnp.maximum(cnt[...], 1.0)
        out_ref[...] = (
            jnp.dot(pooled, wl_ref[...],
                    preferred_element_type=jnp.float32, precision=_HIGH)
            + bl_ref[...]
        )


def _tc_pre(x, w1, degp):
    grid = _N // _BLK
    return pl.pallas_call(
        _tc_pre_body,
        grid=(grid,),
        in_specs=[
            pl.BlockSpec((_BLK, _D), lambda i: (i, 0)),
            pl.BlockSpec((_D, _H), lambda i: (0, 0)),
            pl.BlockSpec((_BLK, _NW), lambda i: (i, 0)),
        ],
        out_specs=[
            pl.BlockSpec((_BLK, _H), lambda i: (i, 0)),
            pl.BlockSpec((_BLK, _H), lambda i: (i, 0)),
            pl.BlockSpec((_BLK, 1), lambda i: (i, 0)),
        ],
        out_shape=[
            jax.ShapeDtypeStruct((_N, _H), jnp.float32),
            jax.ShapeDtypeStruct((_N, _H), jnp.float32),
            jax.ShapeDtypeStruct((_N, 1), jnp.float32),
        ],
    )(x, w1, degp)


def _tc_mid(accp, xw1, dis, b1, w2):
    grid = _N // _BLK
    return pl.pallas_call(
        _tc_mid_body,
        grid=(grid,),
        in_specs=[
            pl.BlockSpec((_NC, _BLK, _H), lambda i: (0, i, 0)),
            pl.BlockSpec((_BLK, _H), lambda i: (i, 0)),
            pl.BlockSpec((_BLK, 1), lambda i: (i, 0)),
            pl.BlockSpec((1, _H), lambda i: (0, 0)),
            pl.BlockSpec((_H, _H), lambda i: (0, 0)),
        ],
        out_specs=[
            pl.BlockSpec((_BLK, _H), lambda i: (i, 0)),
            pl.BlockSpec((_BLK, _H), lambda i: (i, 0)),
        ],
        out_shape=[
            jax.ShapeDtypeStruct((_N, _H), jnp.float32),
            jax.ShapeDtypeStruct((_N, _H), jnp.float32),
        ],
    )(accp, xw1, dis, b1, w2)


def _tc_post(accp, xw2, dis, b2, batch, wl, bl):
    grid = _N // _BLK
    return pl.pallas_call(
        _tc_post_body,
        grid=(grid,),
        in_specs=[
            pl.BlockSpec((_NC, _BLK, _H), lambda i: (0, i, 0)),
            pl.BlockSpec((_BLK, _H), lambda i: (i, 0)),
            pl.BlockSpec((_BLK, 1), lambda i: (i, 0)),
            pl.BlockSpec((1, _H), lambda i: (0, 0)),
            pl.BlockSpec((_BLK, 1), lambda i: (i, 0)),
            pl.BlockSpec((_H, 1), lambda i: (0, 0)),
            pl.BlockSpec((1, 1), lambda i: (0, 0)),
        ],
        out_specs=pl.BlockSpec((_G, 1), lambda i: (0, 0)),
        out_shape=jax.ShapeDtypeStruct((_G, 1), jnp.float32),
        scratch_shapes=[
            pltpu.VMEM((_G, _H), jnp.float32),
            pltpu.VMEM((_G, 1), jnp.float32),
        ],
    )(accp, xw2, dis, b2, batch, wl, bl)


def kernel(x, edge_index, edge_attr, batch, W1, b1, W2, b2, Wl, bl):
    pad = _EPAD - _E
    r = jnp.concatenate([edge_index[0], jnp.zeros((pad,), jnp.int32)])
    c = jnp.concatenate([edge_index[1], jnp.zeros((pad,), jnp.int32)])
    w = jnp.concatenate([edge_attr, jnp.zeros((pad,), jnp.float32)])
    r3 = r.reshape(_NW, _CHUNKS, _CHUNK)
    c3 = c.reshape(_NW, _CHUNKS, _CHUNK)
    w3 = w.reshape(_NW, _CHUNKS, _CHUNK)

    sc_deg, sc_conv = _sc_kernels()
    degp = sc_deg(c3, w3).reshape(_NW, _N).T
    xw1, u1, dis = _tc_pre(x, W1, degp)
    acc1 = sc_conv(u1, r3, c3, w3)
    xw2, u2 = _tc_mid(acc1, xw1, dis, b1.reshape(1, _H), W2)
    acc2 = sc_conv(u2, r3, c3, w3)
    out = _tc_post(acc2, xw2, dis, b2.reshape(1, _H), batch.reshape(_N, 1),
                   Wl, bl.reshape(1, 1))
    return out


# speedup vs baseline: 14.4308x; 1.2638x over previous
import functools

import jax
import jax.numpy as jnp
from jax import lax
from jax.experimental import pallas as pl
from jax.experimental.pallas import tpu as pltpu
from jax.experimental.pallas import tpu_sc as plsc

_N = 10000
_E = 320000
_D = 128
_H = 64
_G = 64

_NC = 2
_NS = 16
_NW = _NC * _NS
_CHUNK = 128
_CHUNKS = 80
_EPT = _CHUNKS * _CHUNK
_EPAD = _EPT * _NW

_RPT = _N // _NS

_BLK = 2000
_HIGH = lax.Precision.HIGHEST


def _sc_deg_body(c_hbm, w_hbm, degp_hbm, cbuf, wbuf, degl):
    cid = lax.axis_index("c")
    sid = lax.axis_index("s")
    wid = cid * _NS + sid
    pltpu.sync_copy(c_hbm.at[wid], cbuf)
    pltpu.sync_copy(w_hbm.at[wid], wbuf)

    z16 = jnp.zeros((16,), jnp.float32)

    @pl.loop(0, _N // 16)
    def _(i):
        degl[i, :] = z16

    @pl.loop(0, _CHUNKS)
    def _(j):
        for k in range(_CHUNK // 16):
            cv = cbuf[j, pl.ds(k * 16, 16)]
            wv = wbuf[j, pl.ds(k * 16, 16)]
            plsc.addupdate_scatter(degl, [cv >> 4, cv & 15], wv)

    pltpu.sync_copy(degl, degp_hbm.at[wid])


@functools.lru_cache(maxsize=None)
def _sc_kernels():
    mesh = plsc.VectorSubcoreMesh(
        core_axis_name="c", subcore_axis_name="s",
        num_cores=_NC, num_subcores=_NS,
    )
    sc_deg = pl.kernel(
        _sc_deg_body,
        out_type=jax.ShapeDtypeStruct((_NW, _N // 16, 16), jnp.float32),
        mesh=mesh,
        scratch_types=[
            pltpu.VMEM((_CHUNKS, _CHUNK), jnp.int32),
            pltpu.VMEM((_CHUNKS, _CHUNK), jnp.float32),
            pltpu.VMEM((_N // 16, 16), jnp.float32),
        ],
        compiler_params=pltpu.CompilerParams(needs_layout_passes=False, use_tc_tiling_on_sc=False),
    )
    sc_conv = pl.kernel(
        _sc_conv_body,
        out_type=jax.ShapeDtypeStruct((_NC, _N, _H), jnp.float32),
        mesh=mesh,
        scratch_types=[
            pltpu.VMEM((_CHUNKS, _CHUNK), jnp.int32),
            pltpu.VMEM((_CHUNKS, _CHUNK), jnp.int32),
            pltpu.VMEM((_CHUNKS, _CHUNK), jnp.float32),
            pltpu.VMEM((_CHUNK, _H), jnp.float32),
            pltpu.VMEM((_CHUNK, _H), jnp.float32),
            pltpu.VMEM((_CHUNK, _H), jnp.float32),
            pltpu.VMEM((_CHUNK, _H), jnp.float32),
            pltpu.VMEM_SHARED((_N, _H), jnp.float32),
            pltpu.SemaphoreType.DMA,
            pltpu.SemaphoreType.DMA,
            pltpu.SemaphoreType.DMA,
            pltpu.SemaphoreType.DMA,
        ],
        compiler_params=pltpu.CompilerParams(needs_layout_passes=False, use_tc_tiling_on_sc=False),
    )
    return sc_deg, sc_conv


def _sc_conv_body(u_hbm, r_hbm, c_hbm, w_hbm, accp_hbm,
                  rbuf, cbuf, wbuf, rows0, rows1, scb0, scb1, acc_sh,
                  sg0, sg1, ss0, ss1):
    cid = lax.axis_index("c")
    sid = lax.axis_index("s")
    wid = cid * _NS + sid
    pltpu.sync_copy(r_hbm.at[wid], rbuf)
    pltpu.sync_copy(c_hbm.at[wid], cbuf)
    pltpu.sync_copy(w_hbm.at[wid], wbuf)

    z16 = jnp.zeros((16,), jnp.float32)

    @pl.loop(0, _CHUNK)
    def _(i):
        for q in range(_H // 16):
            rows0[i, pl.ds(q * 16, 16)] = z16

    base = sid * _RPT
    for off, nr in ((0, 128), (128, 128), (256, 128), (384, 128), (512, 113)):
        pltpu.sync_copy(rows0.at[pl.ds(0, nr)],
                        acc_sh.at[pl.ds(base + off, nr)])
    plsc.subcore_barrier()

    bufs = ((rows0, scb0, sg0, ss0), (rows1, scb1, sg1, ss1))

    pltpu.async_copy(u_hbm.at[rbuf.at[0]], rows0, sg0)

    @pl.loop(0, _CHUNKS // 2)
    def _(jj):
        for b in range(2):
            rows, scb, sg, ss = bufs[b]
            rown, _, sgn, _ = bufs[1 - b]
            j = jj * 2 + b
            pltpu.make_async_copy(u_hbm.at[rbuf.at[j]], rows, sg).wait()

            @pl.when(j + 1 < _CHUNKS)
            def _():
                pltpu.async_copy(u_hbm.at[rbuf.at[j + 1]], rown, sgn)

            @pl.when(j >= 2)
            def _():
                pltpu.make_async_copy(
                    scb, acc_sh.at[cbuf.at[j - 2]], ss).wait()

            @pl.loop(0, _CHUNK // 16)
            def _(k):
                wv = wbuf[j, pl.ds(k * 16, 16)]
                for e16 in range(16):
                    s = wv[e16]
                    e = k * 16 + e16
                    for q in range(_H // 16):
                        scb[e, pl.ds(q * 16, 16)] = (
                            rows[e, pl.ds(q * 16, 16)] * s)

            pltpu.async_copy(scb, acc_sh.at[cbuf.at[j]], ss, add=True)

    pltpu.make_async_copy(scb0, acc_sh.at[cbuf.at[_CHUNKS - 2]], ss0).wait()
    pltpu.make_async_copy(scb1, acc_sh.at[cbuf.at[_CHUNKS - 1]], ss1).wait()

    plsc.subcore_barrier()
    for off, nr in ((0, 128), (128, 128), (256, 128), (384, 128), (512, 113)):
        sl = pl.ds(base + off, nr)
        pltpu.sync_copy(acc_sh.at[sl], rows0.at[pl.ds(0, nr)])
        pltpu.sync_copy(rows0.at[pl.ds(0, nr)], accp_hbm.at[cid, sl])


def _tc_pre_body(x_ref, w1_ref, degp_ref, xw_ref, u_ref, dis_ref):
    deg = 1.0 + jnp.sum(degp_ref[...], axis=1)
    dis = lax.rsqrt(deg)[:, None]
    xw = jnp.dot(x_ref[...], w1_ref[...],
                 preferred_element_type=jnp.float32, precision=_HIGH)
    xw_ref[...] = xw
    u_ref[...] = xw * dis
    dis_ref[...] = dis


def _tc_mid_body(accp_ref, xw1_ref, dis_ref, b1_ref, w2_ref, xw2_ref, u2_ref):
    dis = dis_ref[...]
    acc = accp_ref[0] + accp_ref[1]
    h1 = jnp.maximum(dis * (acc + dis * xw1_ref[...]) + b1_ref[...], 0.0)
    xw2 = jnp.dot(h1, w2_ref[...],
                  preferred_element_type=jnp.float32, precision=_HIGH)
    xw2_ref[...] = xw2
    u2_ref[...] = xw2 * dis


def _tc_post_body(accp_ref, xw2_ref, dis_ref, b2_ref, batch_ref, wl_ref,
                  bl_ref, out_ref, sums, cnt):
    step = pl.program_id(0)
    dis = dis_ref[...]
    acc = accp_ref[0] + accp_ref[1]
    h2 = dis * (acc + dis * xw2_ref[...]) + b2_ref[...]
    b = batch_ref[...][:, 0]
    gid = lax.broadcasted_iota(jnp.int32, (_G, _BLK), 0)
    m = (b[None, :] == gid).astype(jnp.float32)
    ps = jnp.dot(m, h2, preferred_element_type=jnp.float32, precision=_HIGH)
    pc = jnp.sum(m, axis=1, keepdims=True)

    @pl.when(step == 0)
    def _():
        sums[...] = ps
        cnt[...] = pc

    @pl.when(step > 0)
    def _():
        sums[...] += ps
        cnt[...] += pc

    @pl.when(step == pl.num_programs(0) - 1)
    def _():
        pooled = sums[...] / jnp.maximum(cnt[...], 1.0)
        out_ref[...] = (
            jnp.dot(pooled, wl_ref[...],
                    preferred_element_type=jnp.float32, precision=_HIGH)
            + bl_ref[...]
        )


def _tc_pre(x, w1, degp):
    grid = _N // _BLK
    return pl.pallas_call(
        _tc_pre_body,
        grid=(grid,),
        in_specs=[
            pl.BlockSpec((_BLK, _D), lambda i: (i, 0)),
            pl.BlockSpec((_D, _H), lambda i: (0, 0)),
            pl.BlockSpec((_BLK, _NW), lambda i: (i, 0)),
        ],
        out_specs=[
            pl.BlockSpec((_BLK, _H), lambda i: (i, 0)),
            pl.BlockSpec((_BLK, _H), lambda i: (i, 0)),
            pl.BlockSpec((_BLK, 1), lambda i: (i, 0)),
        ],
        out_shape=[
            jax.ShapeDtypeStruct((_N, _H), jnp.float32),
            jax.ShapeDtypeStruct((_N, _H), jnp.float32),
            jax.ShapeDtypeStruct((_N, 1), jnp.float32),
        ],
    )(x, w1, degp)


def _tc_mid(accp, xw1, dis, b1, w2):
    grid = _N // _BLK
    return pl.pallas_call(
        _tc_mid_body,
        grid=(grid,),
        in_specs=[
            pl.BlockSpec((_NC, _BLK, _H), lambda i: (0, i, 0)),
            pl.BlockSpec((_BLK, _H), lambda i: (i, 0)),
            pl.BlockSpec((_BLK, 1), lambda i: (i, 0)),
            pl.BlockSpec((1, _H), lambda i: (0, 0)),
            pl.BlockSpec((_H, _H), lambda i: (0, 0)),
        ],
        out_specs=[
            pl.BlockSpec((_BLK, _H), lambda i: (i, 0)),
            pl.BlockSpec((_BLK, _H), lambda i: (i, 0)),
        ],
        out_shape=[
            jax.ShapeDtypeStruct((_N, _H), jnp.float32),
            jax.ShapeDtypeStruct((_N, _H), jnp.float32),
        ],
    )(accp, xw1, dis, b1, w2)


def _tc_post(accp, xw2, dis, b2, batch, wl, bl):
    grid = _N // _BLK
    return pl.pallas_call(
        _tc_post_body,
        grid=(grid,),
        in_specs=[
            pl.BlockSpec((_NC, _BLK, _H), lambda i: (0, i, 0)),
            pl.BlockSpec((_BLK, _H), lambda i: (i, 0)),
            pl.BlockSpec((_BLK, 1), lambda i: (i, 0)),
            pl.BlockSpec((1, _H), lambda i: (0, 0)),
            pl.BlockSpec((_BLK, 1), lambda i: (i, 0)),
            pl.BlockSpec((_H, 1), lambda i: (0, 0)),
            pl.BlockSpec((1, 1), lambda i: (0, 0)),
        ],
        out_specs=pl.BlockSpec((_G, 1), lambda i: (0, 0)),
        out_shape=jax.ShapeDtypeStruct((_G, 1), jnp.float32),
        scratch_shapes=[
            pltpu.VMEM((_G, _H), jnp.float32),
            pltpu.VMEM((_G, 1), jnp.float32),
        ],
    )(accp, xw2, dis, b2, batch, wl, bl)


def kernel(x, edge_index, edge_attr, batch, W1, b1, W2, b2, Wl, bl):
    pad = _EPAD - _E
    r = jnp.concatenate([edge_index[0], jnp.zeros((pad,), jnp.int32)])
    c = jnp.concatenate([edge_index[1], jnp.zeros((pad,), jnp.int32)])
    w = jnp.concatenate([edge_attr, jnp.zeros((pad,), jnp.float32)])
    r3 = r.reshape(_NW, _CHUNKS, _CHUNK)
    c3 = c.reshape(_NW, _CHUNKS, _CHUNK)
    w3 = w.reshape(_NW, _CHUNKS, _CHUNK)

    sc_deg, sc_conv = _sc_kernels()
    degp = sc_deg(c3, w3).reshape(_NW, _N).T
    xw1, u1, dis = _tc_pre(x, W1, degp)
    acc1 = sc_conv(u1, r3, c3, w3)
    xw2, u2 = _tc_mid(acc1, xw1, dis, b1.reshape(1, _H), W2)
    acc2 = sc_conv(u2, r3, c3, w3)
    out = _tc_post(acc2, xw2, dis, b2.reshape(1, _H), batch.reshape(_N, 1),
                   Wl, bl.reshape(1, 1))
    return out


# speedup vs baseline: 15.1531x; 1.0501x over previous
import functools

import jax
import jax.numpy as jnp
from jax import lax
from jax.experimental import pallas as pl
from jax.experimental.pallas import tpu as pltpu
from jax.experimental.pallas import tpu_sc as plsc

_N = 10000
_E = 320000
_D = 128
_H = 64
_G = 64

_NC = 2
_NS = 16
_NW = _NC * _NS
_CHUNK = 128
_CHUNKS = 80
_EPT = _CHUNKS * _CHUNK
_EPAD = _EPT * _NW

_RPT = _N // _NS

_BLK = 2000
_HIGH = lax.Precision.HIGHEST


def _sc_deg_body(rc_hbm, w_hbm, degp_hbm, rcbuf, wbuf, degl):
    cid = lax.axis_index("c")
    sid = lax.axis_index("s")
    wid = cid * _NS + sid
    pltpu.sync_copy(rc_hbm.at[wid], rcbuf)
    pltpu.sync_copy(w_hbm.at[wid], wbuf)

    z16 = jnp.zeros((16,), jnp.float32)

    @pl.loop(0, _N // 16)
    def _(i):
        degl[i, :] = z16

    @pl.loop(0, _CHUNKS)
    def _(j):
        for k in range(_CHUNK // 16):
            cv = rcbuf[j, pl.ds(k * 16, 16)] & 16383
            wv = wbuf[j, pl.ds(k * 16, 16)]
            plsc.addupdate_scatter(degl, [cv >> 4, cv & 15], wv)

    pltpu.sync_copy(degl, degp_hbm.at[wid])


@functools.lru_cache(maxsize=None)
def _sc_kernels():
    mesh = plsc.VectorSubcoreMesh(
        core_axis_name="c", subcore_axis_name="s",
        num_cores=_NC, num_subcores=_NS,
    )
    sc_deg = pl.kernel(
        _sc_deg_body,
        out_type=jax.ShapeDtypeStruct((_NW, _N // 16, 16), jnp.float32),
        mesh=mesh,
        scratch_types=[
            pltpu.VMEM((_CHUNKS, _CHUNK), jnp.int32),
            pltpu.VMEM((_CHUNKS, _CHUNK), jnp.float32),
            pltpu.VMEM((_N // 16, 16), jnp.float32),
        ],
        compiler_params=pltpu.CompilerParams(needs_layout_passes=False, use_tc_tiling_on_sc=False),
    )
    sc_conv = pl.kernel(
        _sc_conv_body,
        out_type=jax.ShapeDtypeStruct((_NC, _N, _H), jnp.float32),
        mesh=mesh,
        scratch_types=[
            pltpu.VMEM((_CHUNKS, _CHUNK), jnp.int32),
            pltpu.VMEM((_CHUNKS, _CHUNK), jnp.float32),
            pltpu.VMEM((_CHUNK, _H), jnp.float32),
            pltpu.VMEM((_CHUNK, _H), jnp.float32),
            pltpu.VMEM((_CHUNK, _H), jnp.float32),
            pltpu.VMEM((2, _CHUNK), jnp.int32),
            pltpu.VMEM((1, _CHUNK), jnp.int32),
            pltpu.VMEM_SHARED((_N, _H), jnp.float32),
            pltpu.VMEM_SHARED((_N, _H), jnp.float32),
            pltpu.SemaphoreType.DMA,
            pltpu.SemaphoreType.DMA,
        ],
        compiler_params=pltpu.CompilerParams(needs_layout_passes=False, use_tc_tiling_on_sc=False),
    )
    return sc_deg, sc_conv


def _sc_conv_body(u_hbm, rc_hbm, w_hbm, accp_hbm,
                  rcbuf, wbuf, rows0, rows1, scb, ridx, cidx, acc_sh, u_sh,
                  sg0, sg1):
    cid = lax.axis_index("c")
    sid = lax.axis_index("s")
    wid = cid * _NS + sid
    pltpu.sync_copy(rc_hbm.at[wid], rcbuf)
    pltpu.sync_copy(w_hbm.at[wid], wbuf)

    z16 = jnp.zeros((16,), jnp.float32)

    @pl.loop(0, _CHUNK)
    def _(i):
        for q in range(_H // 16):
            rows0[i, pl.ds(q * 16, 16)] = z16

    base = sid * _RPT
    for off, nr in ((0, 128), (128, 128), (256, 128), (384, 128), (512, 113)):
        pltpu.sync_copy(rows0.at[pl.ds(0, nr)],
                        acc_sh.at[pl.ds(base + off, nr)])
        pltpu.sync_copy(u_hbm.at[pl.ds(base + off, nr)],
                        rows1.at[pl.ds(0, nr)])
        pltpu.sync_copy(rows1.at[pl.ds(0, nr)], u_sh.at[pl.ds(base + off, nr)])
    plsc.subcore_barrier()

    bufs = ((rows0, sg0), (rows1, sg1))

    for k in range(_CHUNK // 16):
        ridx[0, pl.ds(k * 16, 16)] = rcbuf[0, pl.ds(k * 16, 16)] >> 14
    pltpu.async_copy(u_sh.at[ridx.at[0]], rows0, sg0)

    @pl.loop(0, _CHUNKS // 2)
    def _(jj):
        for b in range(2):
            rows, sg = bufs[b]
            rown, sgn = bufs[1 - b]
            j = jj * 2 + b
            pltpu.make_async_copy(u_sh.at[ridx.at[b]], rows, sg).wait()

            @pl.when(j + 1 < _CHUNKS)
            def _():
                for k in range(_CHUNK // 16):
                    ridx[1 - b, pl.ds(k * 16, 16)] = (
                        rcbuf[j + 1, pl.ds(k * 16, 16)] >> 14)
                pltpu.async_copy(u_sh.at[ridx.at[1 - b]], rown, sgn)

            @pl.loop(0, _CHUNK // 16)
            def _(k):
                cidx[0, pl.ds(k * 16, 16)] = (
                    rcbuf[j, pl.ds(k * 16, 16)] & 16383)
                wv = wbuf[j, pl.ds(k * 16, 16)]
                for e16 in range(16):
                    s = wv[e16]
                    e = k * 16 + e16
                    for q in range(_H // 16):
                        scb[e, pl.ds(q * 16, 16)] = (
                            rows[e, pl.ds(q * 16, 16)] * s)

            pltpu.sync_copy(scb, acc_sh.at[cidx.at[0]], add=True)

    plsc.subcore_barrier()
    for off, nr in ((0, 128), (128, 128), (256, 128), (384, 128), (512, 113)):
        sl = pl.ds(base + off, nr)
        pltpu.sync_copy(acc_sh.at[sl], rows0.at[pl.ds(0, nr)])
        pltpu.sync_copy(rows0.at[pl.ds(0, nr)], accp_hbm.at[cid, sl])


def _tc_pre_body(x_ref, w1_ref, degp_ref, xw_ref, u_ref, dis_ref):
    deg = 1.0 + jnp.sum(degp_ref[...], axis=1)
    dis = lax.rsqrt(deg)[:, None]
    xw = jnp.dot(x_ref[...], w1_ref[...],
                 preferred_element_type=jnp.float32, precision=_HIGH)
    xw_ref[...] = xw
    u_ref[...] = xw * dis
    dis_ref[...] = dis


def _tc_mid_body(accp_ref, xw1_ref, dis_ref, b1_ref, w2_ref, xw2_ref, u2_ref):
    dis = dis_ref[...]
    acc = accp_ref[0] + accp_ref[1]
    h1 = jnp.maximum(dis * (acc + dis * xw1_ref[...]) + b1_ref[...], 0.0)
    xw2 = jnp.dot(h1, w2_ref[...],
                  preferred_element_type=jnp.float32, precision=_HIGH)
    xw2_ref[...] = xw2
    u2_ref[...] = xw2 * dis


def _tc_post_body(accp_ref, xw2_ref, dis_ref, b2_ref, batch_ref, wl_ref,
                  bl_ref, out_ref, sums, cnt):
    step = pl.program_id(0)
    dis = dis_ref[...]
    acc = accp_ref[0] + accp_ref[1]
    h2 = dis * (acc + dis * xw2_ref[...]) + b2_ref[...]
    b = batch_ref[...][:, 0]
    gid = lax.broadcasted_iota(jnp.int32, (_G, _BLK), 0)
    m = (b[None, :] == gid).astype(jnp.float32)
    ps = jnp.dot(m, h2, preferred_element_type=jnp.float32, precision=_HIGH)
    pc = jnp.sum(m, axis=1, keepdims=True)

    @pl.when(step == 0)
    def _():
        sums[...] = ps
        cnt[...] = pc

    @pl.when(step > 0)
    def _():
        sums[...] += ps
        cnt[...] += pc

    @pl.when(step == pl.num_programs(0) - 1)
    def _():
        pooled = sums[...] / jnp.maximum(cnt[...], 1.0)
        out_ref[...] = (
            jnp.dot(pooled, wl_ref[...],
                    preferred_element_type=jnp.float32, precision=_HIGH)
            + bl_ref[...]
        )


def _tc_pre(x, w1, degp):
    grid = _N // _BLK
    return pl.pallas_call(
        _tc_pre_body,
        grid=(grid,),
        in_specs=[
            pl.BlockSpec((_BLK, _D), lambda i: (i, 0)),
            pl.BlockSpec((_D, _H), lambda i: (0, 0)),
            pl.BlockSpec((_BLK, _NW), lambda i: (i, 0)),
        ],
        out_specs=[
            pl.BlockSpec((_BLK, _H), lambda i: (i, 0)),
            pl.BlockSpec((_BLK, _H), lambda i: (i, 0)),
            pl.BlockSpec((_BLK, 1), lambda i: (i, 0)),
        ],
        out_shape=[
            jax.ShapeDtypeStruct((_N, _H), jnp.float32),
            jax.ShapeDtypeStruct((_N, _H), jnp.float32),
            jax.ShapeDtypeStruct((_N, 1), jnp.float32),
        ],
    )(x, w1, degp)


def _tc_mid(accp, xw1, dis, b1, w2):
    grid = _N // _BLK
    return pl.pallas_call(
        _tc_mid_body,
        grid=(grid,),
        in_specs=[
            pl.BlockSpec((_NC, _BLK, _H), lambda i: (0, i, 0)),
            pl.BlockSpec((_BLK, _H), lambda i: (i, 0)),
            pl.BlockSpec((_BLK, 1), lambda i: (i, 0)),
            pl.BlockSpec((1, _H), lambda i: (0, 0)),
            pl.BlockSpec((_H, _H), lambda i: (0, 0)),
        ],
        out_specs=[
            pl.BlockSpec((_BLK, _H), lambda i: (i, 0)),
            pl.BlockSpec((_BLK, _H), lambda i: (i, 0)),
        ],
        out_shape=[
            jax.ShapeDtypeStruct((_N, _H), jnp.float32),
            jax.ShapeDtypeStruct((_N, _H), jnp.float32),
        ],
    )(accp, xw1, dis, b1, w2)


def _tc_post(accp, xw2, dis, b2, batch, wl, bl):
    grid = _N // _BLK
    return pl.pallas_call(
        _tc_post_body,
        grid=(grid,),
        in_specs=[
            pl.BlockSpec((_NC, _BLK, _H), lambda i: (0, i, 0)),
            pl.BlockSpec((_BLK, _H), lambda i: (i, 0)),
            pl.BlockSpec((_BLK, 1), lambda i: (i, 0)),
            pl.BlockSpec((1, _H), lambda i: (0, 0)),
            pl.BlockSpec((_BLK, 1), lambda i: (i, 0)),
            pl.BlockSpec((_H, 1), lambda i: (0, 0)),
            pl.BlockSpec((1, 1), lambda i: (0, 0)),
        ],
        out_specs=pl.BlockSpec((_G, 1), lambda i: (0, 0)),
        out_shape=jax.ShapeDtypeStruct((_G, 1), jnp.float32),
        scratch_shapes=[
            pltpu.VMEM((_G, _H), jnp.float32),
            pltpu.VMEM((_G, 1), jnp.float32),
        ],
    )(accp, xw2, dis, b2, batch, wl, bl)


def kernel(x, edge_index, edge_attr, batch, W1, b1, W2, b2, Wl, bl):
    pad = _EPAD - _E
    rc = edge_index[0] * 16384 + edge_index[1]
    rc3 = jnp.concatenate([rc, jnp.zeros((pad,), jnp.int32)]).reshape(
        _NW, _CHUNKS, _CHUNK)
    w3 = jnp.concatenate([edge_attr, jnp.zeros((pad,), jnp.float32)]).reshape(
        _NW, _CHUNKS, _CHUNK)

    sc_deg, sc_conv = _sc_kernels()
    degp = sc_deg(rc3, w3).reshape(_NW, _N).T
    xw1, u1, dis = _tc_pre(x, W1, degp)
    acc1 = sc_conv(u1, rc3, w3)
    xw2, u2 = _tc_mid(acc1, xw1, dis, b1.reshape(1, _H), W2)
    acc2 = sc_conv(u2, rc3, w3)
    out = _tc_post(acc2, xw2, dis, b2.reshape(1, _H), batch.reshape(_N, 1),
                   Wl, bl.reshape(1, 1))
    return out


# speedup vs baseline: 24.9738x; 1.6481x over previous
import functools

import jax
import jax.numpy as jnp
from jax import lax
from jax.experimental import pallas as pl
from jax.experimental.pallas import tpu as pltpu
from jax.experimental.pallas import tpu_sc as plsc

_N = 10000
_E = 320000
_D = 128
_H = 64
_G = 64

_NC = 2
_NS = 16
_NW = _NC * _NS
_CHUNK = 128
_CHUNKS = 80
_EPT = _CHUNKS * _CHUNK
_EPAD = _EPT * _NW

_RPT = _N // _NS

_BLK = 2000
_HIGH = lax.Precision.HIGHEST


def _sc_deg_body(rc_hbm, w_hbm, degp_hbm, rcbuf, wbuf, degl):
    cid = lax.axis_index("c")
    sid = lax.axis_index("s")
    wid = cid * _NS + sid
    pltpu.sync_copy(rc_hbm.at[wid], rcbuf)
    pltpu.sync_copy(w_hbm.at[wid], wbuf)

    z16 = jnp.zeros((16,), jnp.float32)

    @pl.loop(0, _N // 16)
    def _(i):
        degl[i, :] = z16

    @pl.loop(0, _CHUNKS)
    def _(j):
        for k in range(_CHUNK // 16):
            cv = rcbuf[j, pl.ds(k * 16, 16)] & 16383
            wv = wbuf[j, pl.ds(k * 16, 16)]
            plsc.addupdate_scatter(degl, [cv >> 4, cv & 15], wv)

    pltpu.sync_copy(degl, degp_hbm.at[wid])


@functools.lru_cache(maxsize=None)
def _sc_kernels():
    mesh = plsc.VectorSubcoreMesh(
        core_axis_name="c", subcore_axis_name="s",
        num_cores=_NC, num_subcores=_NS,
    )
    sc_deg = pl.kernel(
        _sc_deg_body,
        out_type=jax.ShapeDtypeStruct((_NW, _N // 16, 16), jnp.float32),
        mesh=mesh,
        scratch_types=[
            pltpu.VMEM((_CHUNKS, _CHUNK), jnp.int32),
            pltpu.VMEM((_CHUNKS, _CHUNK), jnp.float32),
            pltpu.VMEM((_N // 16, 16), jnp.float32),
        ],
        compiler_params=pltpu.CompilerParams(needs_layout_passes=False, use_tc_tiling_on_sc=False),
    )
    sc_conv = pl.kernel(
        _sc_conv_body,
        out_type=jax.ShapeDtypeStruct((_NC, _N, _H), jnp.float32),
        mesh=mesh,
        scratch_types=[
            pltpu.VMEM((_CHUNKS, _CHUNK), jnp.int32),
            pltpu.VMEM((_CHUNKS, _CHUNK), jnp.float32),
            pltpu.VMEM((_CHUNK, _H), jnp.float32),
            pltpu.VMEM((_CHUNK, _H), jnp.float32),
            pltpu.VMEM((_CHUNK, _H), jnp.float32),
            pltpu.VMEM((2, _CHUNK), jnp.int32),
            pltpu.VMEM((1, _CHUNK), jnp.int32),
            pltpu.VMEM_SHARED((_N, _H), jnp.float32),
            pltpu.VMEM_SHARED((_N, _H), jnp.float32),
            pltpu.SemaphoreType.DMA,
            pltpu.SemaphoreType.DMA,
        ],
        compiler_params=pltpu.CompilerParams(needs_layout_passes=False, use_tc_tiling_on_sc=False),
    )
    return sc_deg, sc_conv


def _sc_conv_body(u_hbm, rc_hbm, w_hbm, accp_hbm,
                  rcbuf, wbuf, rows0, rows1, scb, ridx, cidx, acc_sh, u_sh,
                  sg0, sg1):
    cid = lax.axis_index("c")
    sid = lax.axis_index("s")
    wid = cid * _NS + sid
    pltpu.sync_copy(rc_hbm.at[wid], rcbuf)
    pltpu.sync_copy(w_hbm.at[wid], wbuf)

    z16 = jnp.zeros((16,), jnp.float32)

    @pl.loop(0, _CHUNK)
    def _(i):
        for q in range(_H // 16):
            rows0[i, pl.ds(q * 16, 16)] = z16

    base = sid * _RPT
    for off, nr in ((0, 128), (128, 128), (256, 128), (384, 128), (512, 113)):
        pltpu.sync_copy(rows0.at[pl.ds(0, nr)],
                        acc_sh.at[pl.ds(base + off, nr)])
        pltpu.sync_copy(u_hbm.at[pl.ds(base + off, nr)],
                        rows1.at[pl.ds(0, nr)])
        pltpu.sync_copy(rows1.at[pl.ds(0, nr)], u_sh.at[pl.ds(base + off, nr)])
    plsc.subcore_barrier()

    bufs = ((rows0, sg0), (rows1, sg1))

    for k in range(_CHUNK // 16):
        ridx[0, pl.ds(k * 16, 16)] = rcbuf[0, pl.ds(k * 16, 16)] >> 14
    pltpu.async_copy(u_sh.at[ridx.at[0]], rows0, sg0)

    @pl.loop(0, _CHUNKS // 2)
    def _(jj):
        for b in range(2):
            rows, sg = bufs[b]
            rown, sgn = bufs[1 - b]
            j = jj * 2 + b
            pltpu.make_async_copy(u_sh.at[ridx.at[b]], rows, sg).wait()

            @pl.when(j + 1 < _CHUNKS)
            def _():
                for k in range(_CHUNK // 16):
                    ridx[1 - b, pl.ds(k * 16, 16)] = (
                        rcbuf[j + 1, pl.ds(k * 16, 16)] >> 14)
                pltpu.async_copy(u_sh.at[ridx.at[1 - b]], rown, sgn)

            @plsc.parallel_loop(0, _CHUNK // 16, unroll=2)
            def _(k):
                cidx[0, pl.ds(k * 16, 16)] = (
                    rcbuf[j, pl.ds(k * 16, 16)] & 16383)
                wv = wbuf[j, pl.ds(k * 16, 16)]
                for e16 in range(16):
                    s = wv[e16]
                    e = k * 16 + e16
                    for q in range(_H // 16):
                        scb[e, pl.ds(q * 16, 16)] = (
                            rows[e, pl.ds(q * 16, 16)] * s)

            pltpu.sync_copy(scb, acc_sh.at[cidx.at[0]], add=True)

    plsc.subcore_barrier()
    for off, nr in ((0, 128), (128, 128), (256, 128), (384, 128), (512, 113)):
        sl = pl.ds(base + off, nr)
        pltpu.sync_copy(acc_sh.at[sl], rows0.at[pl.ds(0, nr)])
        pltpu.sync_copy(rows0.at[pl.ds(0, nr)], accp_hbm.at[cid, sl])


def _tc_pre_body(x_ref, w1_ref, degp_ref, xw_ref, u_ref, dis_ref):
    deg = 1.0 + jnp.sum(degp_ref[...], axis=1)
    dis = lax.rsqrt(deg)[:, None]
    xw = jnp.dot(x_ref[...], w1_ref[...],
                 preferred_element_type=jnp.float32, precision=_HIGH)
    xw_ref[...] = xw
    u_ref[...] = xw * dis
    dis_ref[...] = dis


def _tc_mid_body(accp_ref, xw1_ref, dis_ref, b1_ref, w2_ref, xw2_ref, u2_ref):
    dis = dis_ref[...]
    acc = accp_ref[0] + accp_ref[1]
    h1 = jnp.maximum(dis * (acc + dis * xw1_ref[...]) + b1_ref[...], 0.0)
    xw2 = jnp.dot(h1, w2_ref[...],
                  preferred_element_type=jnp.float32, precision=_HIGH)
    xw2_ref[...] = xw2
    u2_ref[...] = xw2 * dis


def _tc_post_body(accp_ref, xw2_ref, dis_ref, b2_ref, batch_ref, wl_ref,
                  bl_ref, out_ref, sums, cnt):
    step = pl.program_id(0)
    dis = dis_ref[...]
    acc = accp_ref[0] + accp_ref[1]
    h2 = dis * (acc + dis * xw2_ref[...]) + b2_ref[...]
    b = batch_ref[...][:, 0]
    gid = lax.broadcasted_iota(jnp.int32, (_G, _BLK), 0)
    m = (b[None, :] == gid).astype(jnp.float32)
    ps = jnp.dot(m, h2, preferred_element_type=jnp.float32, precision=_HIGH)
    pc = jnp.sum(m, axis=1, keepdims=True)

    @pl.when(step == 0)
    def _():
        sums[...] = ps
        cnt[...] = pc

    @pl.when(step > 0)
    def _():
        sums[...] += ps
        cnt[...] += pc

    @pl.when(step == pl.num_programs(0) - 1)
    def _():
        pooled = sums[...] / jnp.maximum(cnt[...], 1.0)
        out_ref[...] = (
            jnp.dot(pooled, wl_ref[...],
                    preferred_element_type=jnp.float32, precision=_HIGH)
            + bl_ref[...]
        )


def _tc_pre(x, w1, degp):
    grid = _N // _BLK
    return pl.pallas_call(
        _tc_pre_body,
        grid=(grid,),
        in_specs=[
            pl.BlockSpec((_BLK, _D), lambda i: (i, 0)),
            pl.BlockSpec((_D, _H), lambda i: (0, 0)),
            pl.BlockSpec((_BLK, _NW), lambda i: (i, 0)),
        ],
        out_specs=[
            pl.BlockSpec((_BLK, _H), lambda i: (i, 0)),
            pl.BlockSpec((_BLK, _H), lambda i: (i, 0)),
            pl.BlockSpec((_BLK, 1), lambda i: (i, 0)),
        ],
        out_shape=[
            jax.ShapeDtypeStruct((_N, _H), jnp.float32),
            jax.ShapeDtypeStruct((_N, _H), jnp.float32),
            jax.ShapeDtypeStruct((_N, 1), jnp.float32),
        ],
    )(x, w1, degp)


def _tc_mid(accp, xw1, dis, b1, w2):
    grid = _N // _BLK
    return pl.pallas_call(
        _tc_mid_body,
        grid=(grid,),
        in_specs=[
            pl.BlockSpec((_NC, _BLK, _H), lambda i: (0, i, 0)),
            pl.BlockSpec((_BLK, _H), lambda i: (i, 0)),
            pl.BlockSpec((_BLK, 1), lambda i: (i, 0)),
            pl.BlockSpec((1, _H), lambda i: (0, 0)),
            pl.BlockSpec((_H, _H), lambda i: (0, 0)),
        ],
        out_specs=[
            pl.BlockSpec((_BLK, _H), lambda i: (i, 0)),
            pl.BlockSpec((_BLK, _H), lambda i: (i, 0)),
        ],
        out_shape=[
            jax.ShapeDtypeStruct((_N, _H), jnp.float32),
            jax.ShapeDtypeStruct((_N, _H), jnp.float32),
        ],
    )(accp, xw1, dis, b1, w2)


def _tc_post(accp, xw2, dis, b2, batch, wl, bl):
    grid = _N // _BLK
    return pl.pallas_call(
        _tc_post_body,
        grid=(grid,),
        in_specs=[
            pl.BlockSpec((_NC, _BLK, _H), lambda i: (0, i, 0)),
            pl.BlockSpec((_BLK, _H), lambda i: (i, 0)),
            pl.BlockSpec((_BLK, 1), lambda i: (i, 0)),
            pl.BlockSpec((1, _H), lambda i: (0, 0)),
            pl.BlockSpec((_BLK, 1), lambda i: (i, 0)),
            pl.BlockSpec((_H, 1), lambda i: (0, 0)),
            pl.BlockSpec((1, 1), lambda i: (0, 0)),
        ],
        out_specs=pl.BlockSpec((_G, 1), lambda i: (0, 0)),
        out_shape=jax.ShapeDtypeStruct((_G, 1), jnp.float32),
        scratch_shapes=[
            pltpu.VMEM((_G, _H), jnp.float32),
            pltpu.VMEM((_G, 1), jnp.float32),
        ],
    )(accp, xw2, dis, b2, batch, wl, bl)


def kernel(x, edge_index, edge_attr, batch, W1, b1, W2, b2, Wl, bl):
    pad = _EPAD - _E
    rc = edge_index[0] * 16384 + edge_index[1]
    rc3 = jnp.concatenate([rc, jnp.zeros((pad,), jnp.int32)]).reshape(
        _NW, _CHUNKS, _CHUNK)
    w3 = jnp.concatenate([edge_attr, jnp.zeros((pad,), jnp.float32)]).reshape(
        _NW, _CHUNKS, _CHUNK)

    sc_deg, sc_conv = _sc_kernels()
    degp = sc_deg(rc3, w3).reshape(_NW, _N).T
    xw1, u1, dis = _tc_pre(x, W1, degp)
    acc1 = sc_conv(u1, rc3, w3)
    xw2, u2 = _tc_mid(acc1, xw1, dis, b1.reshape(1, _H), W2)
    acc2 = sc_conv(u2, rc3, w3)
    out = _tc_post(acc2, xw2, dis, b2.reshape(1, _H), batch.reshape(_N, 1),
                   Wl, bl.reshape(1, 1))
    return out
